# baseline (speedup 1.0000x reference)
import functools

import jax
import jax.numpy as jnp
from jax import lax
from jax.experimental import pallas as pl
from jax.experimental.pallas import tpu as pltpu
from jax.experimental.pallas import tpu_sc as plsc

IN_F = 1024
OUT_F = 1024
BATCH = 4096

B_SC = 2048


def _route_body(w_ref, idx_ref, p_ref):
    w = w_ref[...]
    row_max = jnp.max(w, axis=1, keepdims=True)
    col = lax.broadcasted_iota(jnp.int32, w.shape, 1)
    masked = jnp.where(w == row_max, col, 2**30)
    idx = jnp.min(masked, axis=1, keepdims=True)
    idx_ref[...] = idx
    p_ref[...] = (col == idx).astype(jnp.bfloat16)


def _route(raw_weight):
    return pl.pallas_call(
        _route_body,
        out_shape=[
            jax.ShapeDtypeStruct((OUT_F, 1), jnp.int32),
            jax.ShapeDtypeStruct((OUT_F, IN_F), jnp.bfloat16),
        ],
    )(raw_weight)


_BB = 512


def _mm_body(x_ref, p_ref, o_ref):
    x = x_ref[...]
    hi = x.astype(jnp.bfloat16)
    lo = (x - hi.astype(jnp.float32)).astype(jnp.bfloat16)
    dn = (((1,), (1,)), ((), ()))
    acc = lax.dot_general(hi, p_ref[...], dn,
                          preferred_element_type=jnp.float32)
    acc += lax.dot_general(lo, p_ref[...], dn,
                           preferred_element_type=jnp.float32)
    o_ref[...] = acc


def _mm_gather(x, p_t):
    nblk = (BATCH - B_SC) // _BB
    off = B_SC // _BB
    return pl.pallas_call(
        _mm_body,
        grid=(nblk,),
        in_specs=[
            pl.BlockSpec((_BB, IN_F), lambda i: (off + i, 0)),
            pl.BlockSpec((OUT_F, IN_F), lambda i: (0, 0)),
        ],
        out_specs=pl.BlockSpec((_BB, OUT_F), lambda i: (i, 0)),
        out_shape=jax.ShapeDtypeStruct((BATCH - B_SC, OUT_F), jnp.float32),
    )(x, p_t)


_NC, _NS, _L = 2, 16, 16
_NW = _NC * _NS
_RPW = B_SC // _NW
_R = 16
_NCH = _RPW // _R
_CIDX = IN_F // _L
_NIB = 3


def _sc_gather_body(x_hbm, idx_hbm, out_hbm, idx_v, in_v, out_v,
                    si0, si1, si2, so0, so1):
    wid = lax.axis_index("s") * _NC + lax.axis_index("c")
    base = wid * _RPW
    pltpu.sync_copy(idx_hbm, idx_v)

    in_sems = (si0, si1, si2)
    out_sems = (so0, so1)

    def start_in(g):
        return pltpu.async_copy(
            x_hbm.at[pl.ds(base + g * _R, _R)], in_v.at[g % _NIB],
            in_sems[g % _NIB])

    def start_out(g):
        return pltpu.async_copy(
            out_v.at[g % 2], out_hbm.at[pl.ds(base + g * _R, _R)],
            out_sems[g % 2])

    in_copies = {0: start_in(0), 1: start_in(1)}
    out_copies = {}
    for g in range(_NCH):
        if g + 2 < _NCH:
            in_copies[g + 2] = start_in(g + 2)
        in_copies[g].wait()
        if g >= 2:
            out_copies[g - 2].wait()
        slot = g % _NIB
        oslot = g % 2

        @plsc.parallel_loop(0, _CIDX, unroll=2)
        def cbody(c):
            idxs = idx_v[pl.ds(c * _L, _L)]
            for r in range(_R):
                rows = jnp.full((_L,), r, jnp.int32)
                vals = plsc.load_gather(in_v.at[slot], [rows, idxs])
                out_v[oslot, r, pl.ds(c * _L, _L)] = vals

        out_copies[g] = start_out(g)
    out_copies[_NCH - 2].wait()
    out_copies[_NCH - 1].wait()


def _sc_gather(x, idx):
    mesh = plsc.VectorSubcoreMesh(
        core_axis_name="c", subcore_axis_name="s",
        num_cores=_NC, num_subcores=_NS)
    f = pl.kernel(
        _sc_gather_body,
        out_type=jax.ShapeDtypeStruct((B_SC, OUT_F), jnp.float32),
        mesh=mesh,
        compiler_params=pltpu.CompilerParams(needs_layout_passes=False),
        scratch_types=[
            pltpu.VMEM((IN_F,), jnp.int32),
            pltpu.VMEM((_NIB, _R, IN_F), jnp.float32),
            pltpu.VMEM((2, _R, OUT_F), jnp.float32),
            pltpu.SemaphoreType.DMA,
            pltpu.SemaphoreType.DMA,
            pltpu.SemaphoreType.DMA,
            pltpu.SemaphoreType.DMA,
            pltpu.SemaphoreType.DMA,
        ],
    )
    return f(x, idx)


@jax.jit
def kernel(x, raw_weight):
    idx, p_t = _route(raw_weight)
    out_sc = _sc_gather(x, idx.reshape(IN_F))
    out_tc = _mm_gather(x, p_t)
    return jnp.concatenate([out_sc, out_tc], axis=0)


# speedup vs baseline: 1.2430x; 1.2430x over previous
import functools

import jax
import jax.numpy as jnp
from jax import lax
from jax.experimental import pallas as pl
from jax.experimental.pallas import tpu as pltpu
from jax.experimental.pallas import tpu_sc as plsc

IN_F = 1024
OUT_F = 1024
BATCH = 4096

B_SC = BATCH


def _route_body(w_ref, idx_ref, p_ref):
    w = w_ref[...]
    row_max = jnp.max(w, axis=1, keepdims=True)
    col = lax.broadcasted_iota(jnp.int32, w.shape, 1)
    masked = jnp.where(w == row_max, col, 2**30)
    idx = jnp.min(masked, axis=1, keepdims=True)
    idx_ref[...] = idx
    p_ref[...] = (col == idx).astype(jnp.bfloat16)


def _route(raw_weight):
    return pl.pallas_call(
        _route_body,
        out_shape=[
            jax.ShapeDtypeStruct((OUT_F, 1), jnp.int32),
            jax.ShapeDtypeStruct((OUT_F, IN_F), jnp.bfloat16),
        ],
    )(raw_weight)


_BB = 512


def _mm_body(x_ref, p_ref, o_ref):
    x = x_ref[...]
    hi = x.astype(jnp.bfloat16)
    lo = (x - hi.astype(jnp.float32)).astype(jnp.bfloat16)
    dn = (((1,), (1,)), ((), ()))
    acc = lax.dot_general(hi, p_ref[...], dn,
                          preferred_element_type=jnp.float32)
    acc += lax.dot_general(lo, p_ref[...], dn,
                           preferred_element_type=jnp.float32)
    o_ref[...] = acc


def _mm_gather(x, p_t):
    nblk = (BATCH - B_SC) // _BB
    off = B_SC // _BB
    return pl.pallas_call(
        _mm_body,
        grid=(nblk,),
        in_specs=[
            pl.BlockSpec((_BB, IN_F), lambda i: (off + i, 0)),
            pl.BlockSpec((OUT_F, IN_F), lambda i: (0, 0)),
        ],
        out_specs=pl.BlockSpec((_BB, OUT_F), lambda i: (i, 0)),
        out_shape=jax.ShapeDtypeStruct((BATCH - B_SC, OUT_F), jnp.float32),
    )(x, p_t)


_NC, _NS, _L = 2, 16, 16
_NW = _NC * _NS
_RPW = B_SC // _NW
_R = 16
_NCH = _RPW // _R
_CIDX = IN_F // _L
_NIB = 3
_NOB = 3


def _sc_gather_body(x_hbm, idx_hbm, out_hbm, idx_v, in_v, out_v,
                    si0, si1, si2, so0, so1, so2):
    wid = lax.axis_index("s") * _NC + lax.axis_index("c")
    base = wid * _RPW

    in_sems = (si0, si1, si2)
    out_sems = (so0, so1, so2)

    def start_in(g):
        return pltpu.async_copy(
            x_hbm.at[pl.ds(base + g * _R, _R)], in_v.at[g % _NIB],
            in_sems[g % _NIB])

    def start_out(g):
        return pltpu.async_copy(
            out_v.at[g % _NOB], out_hbm.at[pl.ds(base + g * _R, _R)],
            out_sems[g % _NOB])

    in_copies = {0: start_in(0), 1: start_in(1)}
    pltpu.sync_copy(idx_hbm, idx_v)
    out_copies = {}
    for g in range(_NCH):
        if g + 2 < _NCH:
            in_copies[g + 2] = start_in(g + 2)
        in_copies[g].wait()
        if g >= _NOB:
            out_copies[g - _NOB].wait()
        slot = g % _NIB
        oslot = g % _NOB

        @plsc.parallel_loop(0, _CIDX, unroll=4)
        def cbody(c):
            idxs = idx_v[pl.ds(c * _L, _L)]
            for r in range(_R):
                rows = jnp.full((_L,), r, jnp.int32)
                vals = plsc.load_gather(in_v.at[slot], [rows, idxs])
                out_v[oslot, r, pl.ds(c * _L, _L)] = vals

        out_copies[g] = start_out(g)
    for g in range(max(0, _NCH - _NOB), _NCH):
        out_copies[g].wait()


def _sc_gather(x, idx):
    mesh = plsc.VectorSubcoreMesh(
        core_axis_name="c", subcore_axis_name="s",
        num_cores=_NC, num_subcores=_NS)
    f = pl.kernel(
        _sc_gather_body,
        out_type=jax.ShapeDtypeStruct((B_SC, OUT_F), jnp.float32),
        mesh=mesh,
        compiler_params=pltpu.CompilerParams(needs_layout_passes=False),
        scratch_types=[
            pltpu.VMEM((IN_F,), jnp.int32),
            pltpu.VMEM((_NIB, _R, IN_F), jnp.float32),
            pltpu.VMEM((_NOB, _R, OUT_F), jnp.float32),
            pltpu.SemaphoreType.DMA,
            pltpu.SemaphoreType.DMA,
            pltpu.SemaphoreType.DMA,
            pltpu.SemaphoreType.DMA,
            pltpu.SemaphoreType.DMA,
            pltpu.SemaphoreType.DMA,
        ],
    )
    return f(x, idx)


def _argmax_body(w_ref, idx_ref):
    w = w_ref[...]
    row_max = jnp.max(w, axis=1, keepdims=True)
    col = lax.broadcasted_iota(jnp.int32, w.shape, 1)
    masked = jnp.where(w == row_max, col, 2**30)
    idx_ref[...] = jnp.min(masked, axis=1, keepdims=True)


def _row_argmax(raw_weight):
    return pl.pallas_call(
        _argmax_body,
        out_shape=jax.ShapeDtypeStruct((OUT_F, 1), jnp.int32),
    )(raw_weight)


@jax.jit
def kernel(x, raw_weight):
    idx = _row_argmax(raw_weight)
    return _sc_gather(x, idx.reshape(IN_F))


# speedup vs baseline: 1.2651x; 1.0177x over previous
import functools

import jax
import jax.numpy as jnp
from jax import lax
from jax.experimental import pallas as pl
from jax.experimental.pallas import tpu as pltpu
from jax.experimental.pallas import tpu_sc as plsc

IN_F = 1024
OUT_F = 1024
BATCH = 4096

B_SC = BATCH


def _route_body(w_ref, idx_ref, p_ref):
    w = w_ref[...]
    row_max = jnp.max(w, axis=1, keepdims=True)
    col = lax.broadcasted_iota(jnp.int32, w.shape, 1)
    masked = jnp.where(w == row_max, col, 2**30)
    idx = jnp.min(masked, axis=1, keepdims=True)
    idx_ref[...] = idx
    p_ref[...] = (col == idx).astype(jnp.bfloat16)


def _route(raw_weight):
    return pl.pallas_call(
        _route_body,
        out_shape=[
            jax.ShapeDtypeStruct((OUT_F, 1), jnp.int32),
            jax.ShapeDtypeStruct((OUT_F, IN_F), jnp.bfloat16),
        ],
    )(raw_weight)


_BB = 512


def _mm_body(x_ref, p_ref, o_ref):
    x = x_ref[...]
    hi = x.astype(jnp.bfloat16)
    lo = (x - hi.astype(jnp.float32)).astype(jnp.bfloat16)
    dn = (((1,), (1,)), ((), ()))
    acc = lax.dot_general(hi, p_ref[...], dn,
                          preferred_element_type=jnp.float32)
    acc += lax.dot_general(lo, p_ref[...], dn,
                           preferred_element_type=jnp.float32)
    o_ref[...] = acc


def _mm_gather(x, p_t):
    nblk = (BATCH - B_SC) // _BB
    off = B_SC // _BB
    return pl.pallas_call(
        _mm_body,
        grid=(nblk,),
        in_specs=[
            pl.BlockSpec((_BB, IN_F), lambda i: (off + i, 0)),
            pl.BlockSpec((OUT_F, IN_F), lambda i: (0, 0)),
        ],
        out_specs=pl.BlockSpec((_BB, OUT_F), lambda i: (i, 0)),
        out_shape=jax.ShapeDtypeStruct((BATCH - B_SC, OUT_F), jnp.float32),
    )(x, p_t)


_NC, _NS, _L = 2, 16, 16
_NW = _NC * _NS
_RPW = B_SC // _NW
_R = 16
_NCH = _RPW // _R
_CIDX = IN_F // _L
_NIB = 4
_NOB = 3


def _sc_gather_body(x_hbm, idx_hbm, out_hbm, idx_v, in_v, out_v,
                    si0, si1, si2, si3, so0, so1, so2):
    wid = lax.axis_index("s") * _NC + lax.axis_index("c")
    base = wid * _RPW

    in_sems = (si0, si1, si2, si3)
    out_sems = (so0, so1, so2)

    def start_in(g):
        return pltpu.async_copy(
            x_hbm.at[pl.ds(base + g * _R, _R)], in_v.at[g % _NIB],
            in_sems[g % _NIB])

    def start_out(g):
        return pltpu.async_copy(
            out_v.at[g % _NOB], out_hbm.at[pl.ds(base + g * _R, _R)],
            out_sems[g % _NOB])

    in_copies = {0: start_in(0), 1: start_in(1), 2: start_in(2)}
    pltpu.sync_copy(idx_hbm, idx_v)
    out_copies = {}
    for g in range(_NCH):
        if g + 3 < _NCH:
            in_copies[g + 3] = start_in(g + 3)
        in_copies[g].wait()
        if g >= _NOB:
            out_copies[g - _NOB].wait()
        slot = g % _NIB
        oslot = g % _NOB

        @plsc.parallel_loop(0, _CIDX, unroll=4)
        def cbody(c):
            idxs = idx_v[pl.ds(c * _L, _L)]
            for r in range(_R):
                rows = jnp.full((_L,), r, jnp.int32)
                vals = plsc.load_gather(in_v.at[slot], [rows, idxs])
                out_v[oslot, r, pl.ds(c * _L, _L)] = vals

        out_copies[g] = start_out(g)
    for g in range(max(0, _NCH - _NOB), _NCH):
        out_copies[g].wait()


def _sc_gather(x, idx):
    mesh = plsc.VectorSubcoreMesh(
        core_axis_name="c", subcore_axis_name="s",
        num_cores=_NC, num_subcores=_NS)
    f = pl.kernel(
        _sc_gather_body,
        out_type=jax.ShapeDtypeStruct((B_SC, OUT_F), jnp.float32),
        mesh=mesh,
        compiler_params=pltpu.CompilerParams(needs_layout_passes=False),
        scratch_types=[
            pltpu.VMEM((IN_F,), jnp.int32),
            pltpu.VMEM((_NIB, _R, IN_F), jnp.float32),
            pltpu.VMEM((_NOB, _R, OUT_F), jnp.float32),
            pltpu.SemaphoreType.DMA,
            pltpu.SemaphoreType.DMA,
            pltpu.SemaphoreType.DMA,
            pltpu.SemaphoreType.DMA,
            pltpu.SemaphoreType.DMA,
            pltpu.SemaphoreType.DMA,
            pltpu.SemaphoreType.DMA,
        ],
    )
    return f(x, idx)


def _argmax_body(w_ref, idx_ref):
    w = w_ref[...]
    row_max = jnp.max(w, axis=1, keepdims=True)
    col = lax.broadcasted_iota(jnp.int32, w.shape, 1)
    masked = jnp.where(w == row_max, col, 2**30)
    idx_ref[...] = jnp.min(masked, axis=1, keepdims=True)


def _row_argmax(raw_weight):
    return pl.pallas_call(
        _argmax_body,
        out_shape=jax.ShapeDtypeStruct((OUT_F, 1), jnp.int32),
    )(raw_weight)


@jax.jit
def kernel(x, raw_weight):
    idx = _row_argmax(raw_weight)
    return _sc_gather(x, idx.reshape(IN_F))
